# baseline (device time: 159962 ns/iter reference)
import functools

import jax
import jax.numpy as jnp
from jax import lax
from jax.experimental import pallas as pl
from jax.experimental.pallas import tpu as pltpu

N_DEV = 32


def kernel(x, w_mat):
    k_dim, m_loc = x.shape
    _, n = w_mat.shape
    m_glob = k_dim
    m_per = m_glob // N_DEV

    def body(x_ref, w_ref, out_ref, p_ref, comm_ref, send_sems, recv_sems):
        d = lax.axis_index("i")
        left = (d - 1) % N_DEV
        right = (d + 1) % N_DEV

        barrier_sem = pltpu.get_barrier_semaphore()
        for nbr in (left, right):
            pl.semaphore_signal(
                barrier_sem, inc=1,
                device_id=(nbr,), device_id_type=pl.DeviceIdType.MESH,
            )
        pl.semaphore_wait(barrier_sem, 2)

        p_ref[...] = jnp.dot(
            x_ref[...], w_ref[...], preferred_element_type=jnp.float32
        )

        c0 = (d - 1) % N_DEV
        comm_ref[0, :, :] = p_ref[pl.ds(c0 * m_per, m_per), :].astype(
            jnp.bfloat16
        )

        for h in range(N_DEV - 1):
            rdma = pltpu.make_async_remote_copy(
                src_ref=comm_ref.at[h],
                dst_ref=comm_ref.at[h + 1],
                send_sem=send_sems.at[h],
                recv_sem=recv_sems.at[h],
                device_id=(right,),
                device_id_type=pl.DeviceIdType.MESH,
            )
            rdma.start()
            rdma.wait()

            c = (d - h - 2) % N_DEV
            acc = comm_ref[h + 1, :, :].astype(jnp.float32) + p_ref[
                pl.ds(c * m_per, m_per), :
            ]
            if h < N_DEV - 2:
                comm_ref[h + 1, :, :] = acc.astype(jnp.bfloat16)
            else:
                out_ref[...] = jnp.maximum(acc, 0.0)

        @functools.partial(
            pl.run_scoped, second_barrier=pltpu.SemaphoreType.REGULAR
        )
        def _(second_barrier):
            for nbr in (left, right):
                pl.semaphore_signal(
                    second_barrier, inc=1,
                    device_id=(nbr,), device_id_type=pl.DeviceIdType.MESH,
                )
            pl.semaphore_wait(second_barrier, 2)

    return pl.pallas_call(
        body,
        out_shape=jax.ShapeDtypeStruct((m_per, n), jnp.float32),
        in_specs=[
            pl.BlockSpec(memory_space=pltpu.VMEM),
            pl.BlockSpec(memory_space=pltpu.VMEM),
        ],
        out_specs=pl.BlockSpec(memory_space=pltpu.VMEM),
        scratch_shapes=[
            pltpu.VMEM((m_glob, n), jnp.float32),
            pltpu.VMEM((N_DEV, m_per, n), jnp.bfloat16),
            pltpu.SemaphoreType.DMA((N_DEV - 1,)),
            pltpu.SemaphoreType.DMA((N_DEV - 1,)),
        ],
        compiler_params=pltpu.CompilerParams(collective_id=0),
    )(x, w_mat)


# device time: 127902 ns/iter; 1.2507x vs baseline; 1.2507x over previous
import functools

import jax
import jax.numpy as jnp
from jax import lax
from jax.experimental import pallas as pl
from jax.experimental.pallas import tpu as pltpu

N_DEV = 32
HALF = N_DEV // 2
SUB = 4


def kernel(x, w_mat):
    k_dim, m_loc = x.shape
    _, n = w_mat.shape
    m_glob = k_dim
    m_per = m_glob // N_DEV
    w_sub = n // SUB

    def body(x_ref, w_ref, out_ref, p_ref, r_comm, l_comm,
             r_send, r_recv, l_send, l_recv):
        d = lax.axis_index("i")
        left = (d - 1) % N_DEV
        right = (d + 1) % N_DEV

        barrier_sem = pltpu.get_barrier_semaphore()
        for nbr in (left, right):
            pl.semaphore_signal(
                barrier_sem, inc=1,
                device_id=(nbr,), device_id_type=pl.DeviceIdType.MESH,
            )
        pl.semaphore_wait(barrier_sem, 2)

        p_ref[...] = jnp.dot(
            x_ref[...], w_ref[...], preferred_element_type=jnp.float32
        )

        def pchunk(c, k):
            return p_ref[pl.ds(c * m_per, m_per), k * w_sub:(k + 1) * w_sub]

        cr = (d + HALF) % N_DEV
        cl = (d - HALF + 1) % N_DEV
        for k in range(SUB):
            r_comm[0, k, :, :] = pchunk(cr, k).astype(jnp.bfloat16)
            l_comm[0, k, :, :] = pchunk(cl, k).astype(jnp.bfloat16)

        def mk(comm, sems_send, sems_recv, s, k, dev):
            return pltpu.make_async_remote_copy(
                src_ref=comm.at[s, k],
                dst_ref=comm.at[s + 1, k],
                send_sem=sems_send.at[s, k],
                recv_sem=sems_recv.at[s, k],
                device_id=(dev,),
                device_id_type=pl.DeviceIdType.MESH,
            )

        def step(s, carry):
            for k in range(SUB):
                mk(r_comm, r_send, r_recv, s, k, right).start()

            @pl.when(s < HALF - 1)
            def _():
                for k in range(SUB):
                    mk(l_comm, l_send, l_recv, s, k, left).start()

            for k in range(SUB):
                mk(r_comm, r_send, r_recv, s, k, right).wait_recv()

            @pl.when(s < HALF - 1)
            def _():
                c = (d + HALF - 1 - s) % N_DEV
                for k in range(SUB):
                    acc = r_comm[s + 1, k, :, :].astype(jnp.float32) + pchunk(c, k)
                    r_comm[s + 1, k, :, :] = acc.astype(jnp.bfloat16)

            @pl.when(s < HALF - 1)
            def _():
                for k in range(SUB):
                    mk(l_comm, l_send, l_recv, s, k, left).wait_recv()

            @pl.when(s < HALF - 2)
            def _():
                c = (d - HALF + 2 + s) % N_DEV
                for k in range(SUB):
                    acc = l_comm[s + 1, k, :, :].astype(jnp.float32) + pchunk(c, k)
                    l_comm[s + 1, k, :, :] = acc.astype(jnp.bfloat16)

            return carry

        lax.fori_loop(0, HALF, step, 0)

        for k in range(SUB):
            total = (
                pchunk(d, k)
                + r_comm[HALF, k, :, :].astype(jnp.float32)
                + l_comm[HALF - 1, k, :, :].astype(jnp.float32)
            )
            out_ref[:, k * w_sub:(k + 1) * w_sub] = jnp.maximum(total, 0.0)

        def drain(s, carry):
            for k in range(SUB):
                mk(r_comm, r_send, r_recv, s, k, right).wait_send()

            @pl.when(s < HALF - 1)
            def _():
                for k in range(SUB):
                    mk(l_comm, l_send, l_recv, s, k, left).wait_send()

            return carry

        lax.fori_loop(0, HALF, drain, 0)

        @functools.partial(
            pl.run_scoped, second_barrier=pltpu.SemaphoreType.REGULAR
        )
        def _(second_barrier):
            for nbr in (left, right):
                pl.semaphore_signal(
                    second_barrier, inc=1,
                    device_id=(nbr,), device_id_type=pl.DeviceIdType.MESH,
                )
            pl.semaphore_wait(second_barrier, 2)

    return pl.pallas_call(
        body,
        out_shape=jax.ShapeDtypeStruct((m_per, n), jnp.float32),
        in_specs=[
            pl.BlockSpec(memory_space=pltpu.VMEM),
            pl.BlockSpec(memory_space=pltpu.VMEM),
        ],
        out_specs=pl.BlockSpec(memory_space=pltpu.VMEM),
        scratch_shapes=[
            pltpu.VMEM((m_glob, n), jnp.float32),
            pltpu.VMEM((HALF + 1, SUB, m_per, w_sub), jnp.bfloat16),
            pltpu.VMEM((HALF, SUB, m_per, w_sub), jnp.bfloat16),
            pltpu.SemaphoreType.DMA((HALF, SUB)),
            pltpu.SemaphoreType.DMA((HALF, SUB)),
            pltpu.SemaphoreType.DMA((HALF, SUB)),
            pltpu.SemaphoreType.DMA((HALF, SUB)),
        ],
        compiler_params=pltpu.CompilerParams(collective_id=0),
    )(x, w_mat)


# device time: 88178 ns/iter; 1.8141x vs baseline; 1.4505x over previous
import functools

import jax
import jax.numpy as jnp
import numpy as np
from jax import lax
from jax.experimental import pallas as pl
from jax.experimental.pallas import tpu as pltpu

N_DEV = 32
HALF = N_DEV // 2
SUB = 4


def _cycle_tables():
    identity = np.arange(N_DEV)
    ring = (identity, (identity + 1) % N_DEV, (identity - 1) % N_DEV, identity)
    try:
        import distributed_mesh_v7x as dm

        mesh = dm.get_mesh("i", world_size=N_DEV)
        coords = [tuple(d.coords) for d in mesh.devices.flat]
    except Exception:
        return ring
    xs = sorted({c[0] for c in coords})
    ys = sorted({c[1] for c in coords})
    zs = sorted({c[2] for c in coords})
    if len(coords) != N_DEV or len(xs) != 2 or len(ys) != 4 or len(zs) != 4:
        return ring
    snake = [
        (y, z)
        for y in ys
        for z in (zs if y % 2 == 0 else list(reversed(zs)))
    ]
    cycle = [(xs[0], y, z) for (y, z) in snake] + [
        (xs[1], y, z) for (y, z) in reversed(snake)
    ]
    if set(cycle) != set(coords):
        return ring
    if any(
        sum(abs(a - b) for a, b in zip(cycle[i], cycle[(i + 1) % N_DEV])) != 1
        for i in range(N_DEV)
    ):
        return ring
    log_of = {c: i for i, c in enumerate(coords)}
    ldev = np.array([log_of[c] for c in cycle], dtype=np.int32)
    pos = np.empty(N_DEV, dtype=np.int32)
    pos[ldev] = np.arange(N_DEV, dtype=np.int32)
    nxt = np.empty(N_DEV, dtype=np.int32)
    prv = np.empty(N_DEV, dtype=np.int32)
    for p in range(N_DEV):
        nxt[ldev[p]] = ldev[(p + 1) % N_DEV]
        prv[ldev[p]] = ldev[(p - 1) % N_DEV]
    return pos, nxt, prv, ldev


def kernel(x, w_mat):
    k_dim, m_loc = x.shape
    _, n = w_mat.shape
    m_glob = k_dim
    m_per = m_glob // N_DEV
    w_sub = n // SUB

    pos_t, nxt_t, prv_t, ldev_t = (jnp.asarray(t, jnp.int32) for t in _cycle_tables())

    def body(pos_ref, nxt_ref, prv_ref, ldev_ref, x_ref, w_ref, out_ref,
             p_ref, r_comm, l_comm, r_send, r_recv, l_send, l_recv):
        d = lax.axis_index("i")
        q = pos_ref[d]
        right = nxt_ref[d]
        left = prv_ref[d]

        barrier_sem = pltpu.get_barrier_semaphore()
        for nbr in (left, right):
            pl.semaphore_signal(
                barrier_sem, inc=1,
                device_id=(nbr,), device_id_type=pl.DeviceIdType.MESH,
            )
        pl.semaphore_wait(barrier_sem, 2)

        p_ref[...] = jnp.dot(
            x_ref[...], w_ref[...], preferred_element_type=jnp.float32
        )

        def pchunk(c, k):
            return p_ref[pl.ds(c * m_per, m_per), k * w_sub:(k + 1) * w_sub]

        cr = ldev_ref[(q + HALF) % N_DEV]
        cl = ldev_ref[(q - HALF + 1) % N_DEV]
        for k in range(SUB):
            r_comm[0, k, :, :] = pchunk(cr, k).astype(jnp.bfloat16)
            l_comm[0, k, :, :] = pchunk(cl, k).astype(jnp.bfloat16)

        def mk(comm, sems_send, sems_recv, s, k, dev):
            return pltpu.make_async_remote_copy(
                src_ref=comm.at[s, k],
                dst_ref=comm.at[s + 1, k],
                send_sem=sems_send.at[s, k],
                recv_sem=sems_recv.at[s, k],
                device_id=(dev,),
                device_id_type=pl.DeviceIdType.MESH,
            )

        def step(s, carry):
            for k in range(SUB):
                mk(r_comm, r_send, r_recv, s, k, right).start()

            @pl.when(s < HALF - 1)
            def _():
                for k in range(SUB):
                    mk(l_comm, l_send, l_recv, s, k, left).start()

            for k in range(SUB):
                mk(r_comm, r_send, r_recv, s, k, right).wait_recv()

            @pl.when(s < HALF - 1)
            def _():
                c = ldev_ref[(q + HALF - 1 - s) % N_DEV]
                for k in range(SUB):
                    acc = r_comm[s + 1, k, :, :].astype(jnp.float32) + pchunk(c, k)
                    r_comm[s + 1, k, :, :] = acc.astype(jnp.bfloat16)

            @pl.when(s < HALF - 1)
            def _():
                for k in range(SUB):
                    mk(l_comm, l_send, l_recv, s, k, left).wait_recv()

            @pl.when(s < HALF - 2)
            def _():
                c = ldev_ref[(q - HALF + 2 + s) % N_DEV]
                for k in range(SUB):
                    acc = l_comm[s + 1, k, :, :].astype(jnp.float32) + pchunk(c, k)
                    l_comm[s + 1, k, :, :] = acc.astype(jnp.bfloat16)

            return carry

        lax.fori_loop(0, HALF, step, 0)

        for k in range(SUB):
            total = (
                pchunk(d, k)
                + r_comm[HALF, k, :, :].astype(jnp.float32)
                + l_comm[HALF - 1, k, :, :].astype(jnp.float32)
            )
            out_ref[:, k * w_sub:(k + 1) * w_sub] = jnp.maximum(total, 0.0)

        def drain(s, carry):
            for k in range(SUB):
                mk(r_comm, r_send, r_recv, s, k, right).wait_send()

            @pl.when(s < HALF - 1)
            def _():
                for k in range(SUB):
                    mk(l_comm, l_send, l_recv, s, k, left).wait_send()

            return carry

        lax.fori_loop(0, HALF, drain, 0)

        @functools.partial(
            pl.run_scoped, second_barrier=pltpu.SemaphoreType.REGULAR
        )
        def _(second_barrier):
            for nbr in (left, right):
                pl.semaphore_signal(
                    second_barrier, inc=1,
                    device_id=(nbr,), device_id_type=pl.DeviceIdType.MESH,
                )
            pl.semaphore_wait(second_barrier, 2)

    return pl.pallas_call(
        body,
        out_shape=jax.ShapeDtypeStruct((m_per, n), jnp.float32),
        in_specs=[
            pl.BlockSpec(memory_space=pltpu.SMEM),
            pl.BlockSpec(memory_space=pltpu.SMEM),
            pl.BlockSpec(memory_space=pltpu.SMEM),
            pl.BlockSpec(memory_space=pltpu.SMEM),
            pl.BlockSpec(memory_space=pltpu.VMEM),
            pl.BlockSpec(memory_space=pltpu.VMEM),
        ],
        out_specs=pl.BlockSpec(memory_space=pltpu.VMEM),
        scratch_shapes=[
            pltpu.VMEM((m_glob, n), jnp.float32),
            pltpu.VMEM((HALF + 1, SUB, m_per, w_sub), jnp.bfloat16),
            pltpu.VMEM((HALF, SUB, m_per, w_sub), jnp.bfloat16),
            pltpu.SemaphoreType.DMA((HALF, SUB)),
            pltpu.SemaphoreType.DMA((HALF, SUB)),
            pltpu.SemaphoreType.DMA((HALF, SUB)),
            pltpu.SemaphoreType.DMA((HALF, SUB)),
        ],
        compiler_params=pltpu.CompilerParams(collective_id=0),
    )(pos_t, nxt_t, prv_t, ldev_t, x, w_mat)


# device time: 66611 ns/iter; 2.4014x vs baseline; 1.3238x over previous
import functools

import jax
import jax.numpy as jnp
import numpy as np
from jax import lax
from jax.experimental import pallas as pl
from jax.experimental.pallas import tpu as pltpu

N_DEV = 32
HALF = N_DEV // 2
SUB = 4


def _cycle_tables():
    identity = np.arange(N_DEV)
    ring = (identity, (identity + 1) % N_DEV, (identity - 1) % N_DEV, identity)
    try:
        import distributed_mesh_v7x as dm

        mesh = dm.get_mesh("i", world_size=N_DEV)
        coords = [tuple(d.coords) for d in mesh.devices.flat]
    except Exception:
        return ring
    xs = sorted({c[0] for c in coords})
    ys = sorted({c[1] for c in coords})
    zs = sorted({c[2] for c in coords})
    if len(coords) != N_DEV or len(xs) != 2 or len(ys) != 4 or len(zs) != 4:
        return ring
    snake = [
        (y, z)
        for y in ys
        for z in (zs if y % 2 == 0 else list(reversed(zs)))
    ]
    cycle = [(xs[0], y, z) for (y, z) in snake] + [
        (xs[1], y, z) for (y, z) in reversed(snake)
    ]
    if set(cycle) != set(coords):
        return ring
    if any(
        sum(abs(a - b) for a, b in zip(cycle[i], cycle[(i + 1) % N_DEV])) != 1
        for i in range(N_DEV)
    ):
        return ring
    log_of = {c: i for i, c in enumerate(coords)}
    ldev = np.array([log_of[c] for c in cycle], dtype=np.int32)
    pos = np.empty(N_DEV, dtype=np.int32)
    pos[ldev] = np.arange(N_DEV, dtype=np.int32)
    nxt = np.empty(N_DEV, dtype=np.int32)
    prv = np.empty(N_DEV, dtype=np.int32)
    for p in range(N_DEV):
        nxt[ldev[p]] = ldev[(p + 1) % N_DEV]
        prv[ldev[p]] = ldev[(p - 1) % N_DEV]
    return pos, nxt, prv, ldev


def kernel(x, w_mat):
    k_dim, m_loc = x.shape
    _, n = w_mat.shape
    m_glob = k_dim
    m_per = m_glob // N_DEV
    w_sub = n // SUB

    pos_t, nxt_t, prv_t, ldev_t = (jnp.asarray(t, jnp.int32) for t in _cycle_tables())

    def body(pos_ref, nxt_ref, prv_ref, ldev_ref, x_ref, w_ref, out_ref,
             p_ref, r_comm, l_comm, r_send, r_recv, l_send, l_recv):
        d = lax.axis_index("i")
        q = pos_ref[d]
        right = nxt_ref[d]
        left = prv_ref[d]

        barrier_sem = pltpu.get_barrier_semaphore()
        for nbr in (left, right):
            pl.semaphore_signal(
                barrier_sem, inc=1,
                device_id=(nbr,), device_id_type=pl.DeviceIdType.MESH,
            )
        pl.semaphore_wait(barrier_sem, 2)

        p_ref[...] = jnp.dot(
            x_ref[...], w_ref[...], preferred_element_type=jnp.float32
        )

        def pchunk(c, k):
            return p_ref[pl.ds(c * m_per, m_per), k * w_sub:(k + 1) * w_sub]

        cr = ldev_ref[(q + HALF) % N_DEV]
        cl = ldev_ref[(q - HALF + 1) % N_DEV]
        for k in range(SUB):
            r_comm[0, k, :, :] = pchunk(cr, k).astype(jnp.bfloat16)
            l_comm[0, k, :, :] = pchunk(cl, k).astype(jnp.bfloat16)

        def mk(comm, sems_send, sems_recv, s, k, dev):
            return pltpu.make_async_remote_copy(
                src_ref=comm.at[s, k],
                dst_ref=comm.at[s + 1, k],
                send_sem=sems_send.at[s, k],
                recv_sem=sems_recv.at[s, k],
                device_id=(dev,),
                device_id_type=pl.DeviceIdType.MESH,
            )

        for k in range(SUB):
            mk(r_comm, r_send, r_recv, 0, k, right).start()
            mk(l_comm, l_send, l_recv, 0, k, left).start()

        def slot(t, carry):
            for k in range(SUB):
                s = t - k
                act_r = jnp.logical_and(s >= 0, s < HALF)
                sr = jnp.clip(s, 0, HALF - 1)

                @pl.when(act_r)
                def _():
                    mk(r_comm, r_send, r_recv, sr, k, right).wait_recv()

                @pl.when(jnp.logical_and(act_r, s < HALF - 1))
                def _():
                    c = ldev_ref[(q + HALF - 1 - sr) % N_DEV]
                    acc = r_comm[sr + 1, k, :, :].astype(jnp.float32) + pchunk(c, k)
                    r_comm[sr + 1, k, :, :] = acc.astype(jnp.bfloat16)
                    mk(r_comm, r_send, r_recv, sr + 1, k, right).start()

                act_l = jnp.logical_and(s >= 0, s < HALF - 1)
                sl = jnp.clip(s, 0, HALF - 2)

                @pl.when(act_l)
                def _():
                    mk(l_comm, l_send, l_recv, sl, k, left).wait_recv()

                @pl.when(jnp.logical_and(act_l, s < HALF - 2))
                def _():
                    c = ldev_ref[(q - HALF + 2 + sl) % N_DEV]
                    acc = l_comm[sl + 1, k, :, :].astype(jnp.float32) + pchunk(c, k)
                    l_comm[sl + 1, k, :, :] = acc.astype(jnp.bfloat16)
                    mk(l_comm, l_send, l_recv, sl + 1, k, left).start()

            return carry

        lax.fori_loop(0, HALF + SUB - 1, slot, 0)

        for k in range(SUB):
            total = (
                pchunk(d, k)
                + r_comm[HALF, k, :, :].astype(jnp.float32)
                + l_comm[HALF - 1, k, :, :].astype(jnp.float32)
            )
            out_ref[:, k * w_sub:(k + 1) * w_sub] = jnp.maximum(total, 0.0)

        def drain(s, carry):
            for k in range(SUB):
                mk(r_comm, r_send, r_recv, s, k, right).wait_send()

            @pl.when(s < HALF - 1)
            def _():
                for k in range(SUB):
                    mk(l_comm, l_send, l_recv, s, k, left).wait_send()

            return carry

        lax.fori_loop(0, HALF, drain, 0)

        @functools.partial(
            pl.run_scoped, second_barrier=pltpu.SemaphoreType.REGULAR
        )
        def _(second_barrier):
            for nbr in (left, right):
                pl.semaphore_signal(
                    second_barrier, inc=1,
                    device_id=(nbr,), device_id_type=pl.DeviceIdType.MESH,
                )
            pl.semaphore_wait(second_barrier, 2)

    return pl.pallas_call(
        body,
        out_shape=jax.ShapeDtypeStruct((m_per, n), jnp.float32),
        in_specs=[
            pl.BlockSpec(memory_space=pltpu.SMEM),
            pl.BlockSpec(memory_space=pltpu.SMEM),
            pl.BlockSpec(memory_space=pltpu.SMEM),
            pl.BlockSpec(memory_space=pltpu.SMEM),
            pl.BlockSpec(memory_space=pltpu.VMEM),
            pl.BlockSpec(memory_space=pltpu.VMEM),
        ],
        out_specs=pl.BlockSpec(memory_space=pltpu.VMEM),
        scratch_shapes=[
            pltpu.VMEM((m_glob, n), jnp.float32),
            pltpu.VMEM((HALF + 1, SUB, m_per, w_sub), jnp.bfloat16),
            pltpu.VMEM((HALF, SUB, m_per, w_sub), jnp.bfloat16),
            pltpu.SemaphoreType.DMA((HALF, SUB)),
            pltpu.SemaphoreType.DMA((HALF, SUB)),
            pltpu.SemaphoreType.DMA((HALF, SUB)),
            pltpu.SemaphoreType.DMA((HALF, SUB)),
        ],
        compiler_params=pltpu.CompilerParams(collective_id=0),
    )(pos_t, nxt_t, prv_t, ldev_t, x, w_mat)


# device time: 64361 ns/iter; 2.4854x vs baseline; 1.0350x over previous
import functools

import jax
import jax.numpy as jnp
import numpy as np
from jax import lax
from jax.experimental import pallas as pl
from jax.experimental.pallas import tpu as pltpu

N_DEV = 32
HALF = N_DEV // 2
SUB = 4


def _cycle_tables():
    identity = np.arange(N_DEV)
    ring = (identity, (identity + 1) % N_DEV, (identity - 1) % N_DEV, identity)
    try:
        import distributed_mesh_v7x as dm

        mesh = dm.get_mesh("i", world_size=N_DEV)
        coords = [tuple(d.coords) for d in mesh.devices.flat]
    except Exception:
        return ring
    xs = sorted({c[0] for c in coords})
    ys = sorted({c[1] for c in coords})
    zs = sorted({c[2] for c in coords})
    if len(coords) != N_DEV or len(xs) != 2 or len(ys) != 4 or len(zs) != 4:
        return ring
    snake = [
        (y, z)
        for y in ys
        for z in (zs if y % 2 == 0 else list(reversed(zs)))
    ]
    cycle = [(xs[0], y, z) for (y, z) in snake] + [
        (xs[1], y, z) for (y, z) in reversed(snake)
    ]
    if set(cycle) != set(coords):
        return ring
    if any(
        sum(abs(a - b) for a, b in zip(cycle[i], cycle[(i + 1) % N_DEV])) != 1
        for i in range(N_DEV)
    ):
        return ring
    log_of = {c: i for i, c in enumerate(coords)}
    ldev = np.array([log_of[c] for c in cycle], dtype=np.int32)
    pos = np.empty(N_DEV, dtype=np.int32)
    pos[ldev] = np.arange(N_DEV, dtype=np.int32)
    nxt = np.empty(N_DEV, dtype=np.int32)
    prv = np.empty(N_DEV, dtype=np.int32)
    for p in range(N_DEV):
        nxt[ldev[p]] = ldev[(p + 1) % N_DEV]
        prv[ldev[p]] = ldev[(p - 1) % N_DEV]
    return pos, nxt, prv, ldev


def kernel(x, w_mat):
    k_dim, m_loc = x.shape
    _, n = w_mat.shape
    m_glob = k_dim
    m_per = m_glob // N_DEV
    w_sub = n // SUB

    pos_t, nxt_t, prv_t, ldev_t = (jnp.asarray(t, jnp.int32) for t in _cycle_tables())

    def body(pos_ref, nxt_ref, prv_ref, ldev_ref, x_ref, w_ref, out_ref,
             p_ref, r_comm, l_comm, r_send, r_recv, l_send, l_recv):
        d = lax.axis_index("i")
        q = pos_ref[d]
        right = nxt_ref[d]
        left = prv_ref[d]

        barrier_sem = pltpu.get_barrier_semaphore()
        for nbr in (left, right):
            pl.semaphore_signal(
                barrier_sem, inc=1,
                device_id=(nbr,), device_id_type=pl.DeviceIdType.MESH,
            )
        pl.semaphore_wait(barrier_sem, 2)

        def pchunk(c, k):
            return p_ref[pl.ds(c * m_per, m_per), k * w_sub:(k + 1) * w_sub]

        cr = ldev_ref[(q + HALF) % N_DEV]
        cl = ldev_ref[(q - HALF + 1) % N_DEV]
        seed_r = jnp.dot(
            x_ref[pl.ds(cr * m_per, m_per), :], w_ref[...],
            preferred_element_type=jnp.float32,
        ).astype(jnp.bfloat16)
        seed_l = jnp.dot(
            x_ref[pl.ds(cl * m_per, m_per), :], w_ref[...],
            preferred_element_type=jnp.float32,
        ).astype(jnp.bfloat16)
        for k in range(SUB):
            r_comm[0, k, :, :] = seed_r[:, k * w_sub:(k + 1) * w_sub]
            l_comm[0, k, :, :] = seed_l[:, k * w_sub:(k + 1) * w_sub]

        def mk(comm, sems_send, sems_recv, s, k, dev):
            return pltpu.make_async_remote_copy(
                src_ref=comm.at[s, k],
                dst_ref=comm.at[s + 1, k],
                send_sem=sems_send.at[s, k],
                recv_sem=sems_recv.at[s, k],
                device_id=(dev,),
                device_id_type=pl.DeviceIdType.MESH,
            )

        for k in range(SUB):
            mk(r_comm, r_send, r_recv, 0, k, right).start()
            mk(l_comm, l_send, l_recv, 0, k, left).start()

        p_ref[...] = jnp.dot(
            x_ref[...], w_ref[...], preferred_element_type=jnp.float32
        ).astype(jnp.bfloat16)

        def slot(t, carry):
            for k in range(SUB):
                s = t - k
                act_r = jnp.logical_and(s >= 0, s < HALF)
                sr = jnp.clip(s, 0, HALF - 1)

                @pl.when(act_r)
                def _():
                    mk(r_comm, r_send, r_recv, sr, k, right).wait_recv()

                @pl.when(jnp.logical_and(act_r, s < HALF - 1))
                def _():
                    c = ldev_ref[(q + HALF - 1 - sr) % N_DEV]
                    r_comm[sr + 1, k, :, :] = r_comm[sr + 1, k, :, :] + pchunk(c, k)
                    mk(r_comm, r_send, r_recv, sr + 1, k, right).start()

                act_l = jnp.logical_and(s >= 0, s < HALF - 1)
                sl = jnp.clip(s, 0, HALF - 2)

                @pl.when(act_l)
                def _():
                    mk(l_comm, l_send, l_recv, sl, k, left).wait_recv()

                @pl.when(jnp.logical_and(act_l, s < HALF - 2))
                def _():
                    c = ldev_ref[(q - HALF + 2 + sl) % N_DEV]
                    l_comm[sl + 1, k, :, :] = l_comm[sl + 1, k, :, :] + pchunk(c, k)
                    mk(l_comm, l_send, l_recv, sl + 1, k, left).start()

            return carry

        lax.fori_loop(0, HALF + SUB - 1, slot, 0)

        for k in range(SUB):
            total = (
                pchunk(d, k).astype(jnp.float32)
                + r_comm[HALF, k, :, :].astype(jnp.float32)
                + l_comm[HALF - 1, k, :, :].astype(jnp.float32)
            )
            out_ref[:, k * w_sub:(k + 1) * w_sub] = jnp.maximum(total, 0.0)

        def drain(s, carry):
            for k in range(SUB):
                mk(r_comm, r_send, r_recv, s, k, right).wait_send()

            @pl.when(s < HALF - 1)
            def _():
                for k in range(SUB):
                    mk(l_comm, l_send, l_recv, s, k, left).wait_send()

            return carry

        lax.fori_loop(0, HALF, drain, 0)

        @functools.partial(
            pl.run_scoped, second_barrier=pltpu.SemaphoreType.REGULAR
        )
        def _(second_barrier):
            for nbr in (left, right):
                pl.semaphore_signal(
                    second_barrier, inc=1,
                    device_id=(nbr,), device_id_type=pl.DeviceIdType.MESH,
                )
            pl.semaphore_wait(second_barrier, 2)

    return pl.pallas_call(
        body,
        out_shape=jax.ShapeDtypeStruct((m_per, n), jnp.float32),
        in_specs=[
            pl.BlockSpec(memory_space=pltpu.SMEM),
            pl.BlockSpec(memory_space=pltpu.SMEM),
            pl.BlockSpec(memory_space=pltpu.SMEM),
            pl.BlockSpec(memory_space=pltpu.SMEM),
            pl.BlockSpec(memory_space=pltpu.VMEM),
            pl.BlockSpec(memory_space=pltpu.VMEM),
        ],
        out_specs=pl.BlockSpec(memory_space=pltpu.VMEM),
        scratch_shapes=[
            pltpu.VMEM((m_glob, n), jnp.bfloat16),
            pltpu.VMEM((HALF + 1, SUB, m_per, w_sub), jnp.bfloat16),
            pltpu.VMEM((HALF, SUB, m_per, w_sub), jnp.bfloat16),
            pltpu.SemaphoreType.DMA((HALF, SUB)),
            pltpu.SemaphoreType.DMA((HALF, SUB)),
            pltpu.SemaphoreType.DMA((HALF, SUB)),
            pltpu.SemaphoreType.DMA((HALF, SUB)),
        ],
        compiler_params=pltpu.CompilerParams(collective_id=0),
    )(pos_t, nxt_t, prv_t, ldev_t, x, w_mat)
